# baseline (speedup 1.0000x reference)
import jax
import jax.numpy as jnp
from jax.experimental import pallas as pl

_B = 4
_D = 64
_NCLS = 16


def _mlp_kernel(x_ref, w1_ref, b1_ref, w2_ref, b2_ref, wo_ref, bo_ref, out_ref):
    x0 = x_ref[:, 0, :]
    old = jnp.dot(x0, w1_ref[...], preferred_element_type=jnp.float32)
    old = jnp.maximum(old + b1_ref[...], 0.0)
    new = jnp.dot(old, w2_ref[...], preferred_element_type=jnp.float32)
    new = jnp.maximum(new + b2_ref[...], 0.0)
    logits = (
        jnp.dot(old, wo_ref[:_D, :], preferred_element_type=jnp.float32)
        + jnp.dot(new, wo_ref[_D:, :], preferred_element_type=jnp.float32)
        + bo_ref[...]
    )
    m = jnp.max(logits, axis=-1, keepdims=True)
    e = jnp.exp(logits - m)
    out_ref[...] = e / jnp.sum(e, axis=-1, keepdims=True)


def kernel(x_feats, edge_index, agg_W, agg_b, W1, b1, W2, b2, W_out, b_out):
    del edge_index, agg_W, agg_b
    B, _, D = x_feats.shape
    H = W1.shape[1]
    C = W_out.shape[1]
    return pl.pallas_call(
        _mlp_kernel,
        grid=(1,),
        in_specs=[
            pl.BlockSpec((B, 8, D), lambda i: (0, 0, 0)),
            pl.BlockSpec((D, H), lambda i: (0, 0)),
            pl.BlockSpec((1, H), lambda i: (0, 0)),
            pl.BlockSpec((H, H), lambda i: (0, 0)),
            pl.BlockSpec((1, H), lambda i: (0, 0)),
            pl.BlockSpec((2 * H, C), lambda i: (0, 0)),
            pl.BlockSpec((1, C), lambda i: (0, 0)),
        ],
        out_specs=pl.BlockSpec((B, C), lambda i: (0, 0)),
        out_shape=jax.ShapeDtypeStruct((B, C), jnp.float32),
    )(
        x_feats,
        W1,
        b1.reshape(1, H),
        W2,
        b2.reshape(1, H),
        W_out,
        b_out.reshape(1, C),
    )


# speedup vs baseline: 3.7762x; 3.7762x over previous
import jax
import jax.numpy as jnp
from jax.experimental import pallas as pl

_B = 4
_D = 64
_NCLS = 16


def _mlp_kernel(x_ref, w1_ref, b1_ref, w2_ref, b2_ref, wo_ref, bo_ref, out_ref):
    x0 = x_ref[...]
    old = jnp.dot(x0, w1_ref[...], preferred_element_type=jnp.float32)
    old = jnp.maximum(old + b1_ref[...], 0.0)
    new = jnp.dot(old, w2_ref[...], preferred_element_type=jnp.float32)
    new = jnp.maximum(new + b2_ref[...], 0.0)
    logits = (
        jnp.dot(old, wo_ref[:_D, :], preferred_element_type=jnp.float32)
        + jnp.dot(new, wo_ref[_D:, :], preferred_element_type=jnp.float32)
        + bo_ref[...]
    )
    m = jnp.max(logits, axis=-1, keepdims=True)
    e = jnp.exp(logits - m)
    out_ref[...] = e / jnp.sum(e, axis=-1, keepdims=True)


def kernel(x_feats, edge_index, agg_W, agg_b, W1, b1, W2, b2, W_out, b_out):
    del edge_index, agg_W, agg_b
    B, _, D = x_feats.shape
    H = W1.shape[1]
    C = W_out.shape[1]
    x0 = jax.lax.slice_in_dim(x_feats, 0, 1, axis=1).reshape(B, D)
    return pl.pallas_call(
        _mlp_kernel,
        out_shape=jax.ShapeDtypeStruct((B, C), jnp.float32),
    )(
        x0,
        W1,
        b1.reshape(1, H),
        W2,
        b2.reshape(1, H),
        W_out,
        b_out.reshape(1, C),
    )
